# baseline (speedup 1.0000x reference)
import functools

import jax
import jax.numpy as jnp
from jax import lax
from jax.experimental import pallas as pl
from jax.experimental.pallas import tpu as pltpu
from jax.experimental.pallas import tpu_sc as plsc

N = 7
R = 16
E = 640000
HID = 16
OUT = 8
RS = R * N
BINS = N * RS
L = 16
NW = 16
EPW = E // NW
CH = 8000
NCH = EPW // CH
CVECS = CH // L


def _sc_hist_kernel(ei_hbm, typ_hbm, out_hbm,
                    s0, d0, t0, s1, d1, t1, hist_v, final_v,
                    sem0, sem1):
    wid = lax.axis_index("s")
    base = wid * EPW
    bufs = ((s0, d0, t0), (s1, d1, t1))
    sems = (sem0, sem1)

    def start(c):
        sb, db, tb = bufs[c % 2]
        off = base + c * CH
        sem = sems[c % 2]
        return (pltpu.async_copy(ei_hbm.at[pl.ds(off, CH)], sb, sem),
                pltpu.async_copy(ei_hbm.at[pl.ds(E + off, CH)], db, sem),
                pltpu.async_copy(typ_hbm.at[pl.ds(off, CH)], tb, sem))

    cps = start(0)

    zeros = jnp.zeros((L,), jnp.float32)

    @plsc.parallel_loop(0, BINS, unroll=8)
    def _(i):
        hist_v[pl.ds(i * L, L)] = zeros

    lane_off = lax.broadcasted_iota(jnp.int32, (L,), 0) * BINS
    ones = jnp.ones((L,), jnp.float32)

    def process(sb, db, tb):
        @plsc.parallel_loop(0, CVECS, unroll=8)
        def _(i):
            o = i * L
            s = sb[pl.ds(o, L)]
            d = db[pl.ds(o, L)]
            t = tb[pl.ds(o, L)]
            plsc.addupdate_scatter(
                hist_v, [(d * RS + t * N) + (s + lane_off)], ones)

    for c in range(NCH):
        for cp in cps:
            cp.wait()
        if c + 1 < NCH:
            cps = start(c + 1)
        process(*bufs[c % 2])

    @plsc.parallel_loop(0, BINS // L, unroll=2)
    def _(i):
        o = i * L
        acc = hist_v[pl.ds(o, L)]
        for l in range(1, L):
            acc = acc + hist_v[pl.ds(l * BINS + o, L)]
        final_v[pl.ds(o, L)] = acc

    pltpu.sync_copy(final_v, out_hbm.at[wid])


def _make_sc_hist():
    return pl.kernel(
        _sc_hist_kernel,
        mesh=plsc.VectorSubcoreMesh(core_axis_name="c", subcore_axis_name="s",
                                    num_cores=1),
        out_type=jax.ShapeDtypeStruct((NW, BINS), jnp.float32),
        compiler_params=pltpu.CompilerParams(needs_layout_passes=False),
        scratch_types=[
            pltpu.VMEM((CH,), jnp.int32),
            pltpu.VMEM((CH,), jnp.int32),
            pltpu.VMEM((CH,), jnp.int32),
            pltpu.VMEM((CH,), jnp.int32),
            pltpu.VMEM((CH,), jnp.int32),
            pltpu.VMEM((CH,), jnp.int32),
            pltpu.VMEM((L * BINS,), jnp.float32),
            pltpu.VMEM((BINS,), jnp.float32),
            pltpu.SemaphoreType.DMA,
            pltpu.SemaphoreType.DMA,
        ],
    )


def _tc_finish_body(parts_ref, w1_ref, r1_ref, b1_ref, w2_ref, rt2_ref,
                    b2_ref, out_ref):
    counts = jnp.sum(parts_ref[...], axis=0)
    g = (lax.broadcasted_iota(jnp.int32, (RS, R), 0) // N
         == lax.broadcasted_iota(jnp.int32, (RS, R), 1)).astype(jnp.float32)
    gt = (lax.broadcasted_iota(jnp.int32, (R, RS), 1) // N
          == lax.broadcasted_iota(jnp.int32, (R, RS), 0)).astype(jnp.float32)
    cnt = jnp.dot(counts, g, preferred_element_type=jnp.float32)
    denom = jnp.maximum(
        jnp.dot(cnt, gt, preferred_element_type=jnp.float32), 1.0)
    q = counts / denom
    agg1 = jnp.dot(q, w1_ref[...], preferred_element_type=jnp.float32)
    h = jnp.maximum(agg1 + r1_ref[...] + b1_ref[...], 0.0)
    w2h = jnp.concatenate(
        [jnp.dot(h, w2_ref[r], preferred_element_type=jnp.float32)
         for r in range(R)], axis=0)
    acc = (jnp.dot(q, w2h, preferred_element_type=jnp.float32)
           + jnp.dot(h, rt2_ref[...], preferred_element_type=jnp.float32)
           + b2_ref[...])
    m = jnp.max(acc, axis=1, keepdims=True)
    e = jnp.exp(acc - m)
    lse = jnp.log(jnp.sum(e, axis=1, keepdims=True))
    out_ref[...] = acc - m - lse


def kernel(x, edge_index, edge_type, weight1, root1, bias1, weight2, root2,
           bias2):
    del x
    parts = _make_sc_hist()(edge_index.reshape(2 * E), edge_type)
    parts = parts.reshape(NW, N, RS)
    return pl.pallas_call(
        _tc_finish_body,
        out_shape=jax.ShapeDtypeStruct((N, OUT), jnp.float32),
    )(parts, weight1.reshape(RS, HID), root1, bias1.reshape(1, HID),
      weight2, root2, bias2.reshape(1, OUT))
